# baseline (speedup 1.0000x reference)
import functools

import jax
import jax.numpy as jnp
from jax import lax
from jax.experimental import pallas as pl
from jax.experimental.pallas import tpu as pltpu
from jax.experimental.pallas import tpu_sc as plsc

_N = 20000
_NUM = 2048
_NT = 16
_PPT = 1250
_SL = 79
_PPAD = _SL * 16
_STR = 64


def _sc_body(x_hbm, y_hbm, z_hbm, w_hbm, out_hbm,
             x_v, y_v, z_v, w_v, dm_v, rec_v, all_v,
             o0_v, o1_v, o2_v, o3_v, shared):
    t = lax.axis_index("s")
    inf = jnp.float32(jnp.inf)
    io = lax.iota(jnp.int32, 16)

    pltpu.sync_copy(x_hbm.at[t], x_v)
    pltpu.sync_copy(y_hbm.at[t], y_v)
    pltpu.sync_copy(z_hbm.at[t], z_v)
    pltpu.sync_copy(w_hbm.at[t], w_v)

    def init_dm(s, carry):
        p = s * 16 + io
        dm_v[pl.ds(s * 16, 16)] = jnp.where(p < _PPT, inf, -inf)
        return carry

    lax.fori_loop(0, _SL, init_dm, 0)

    v0 = jnp.where(t == 0, inf, -inf)
    rec0 = jnp.where(io == 0, v0,
           jnp.where(io == 1, jnp.float32(0.0),
           jnp.where(io == 2, x_v[pl.ds(0, 16)][0],
           jnp.where(io == 3, y_v[pl.ds(0, 16)][0],
           jnp.where(io == 4, z_v[pl.ds(0, 16)][0],
                     w_v[pl.ds(0, 16)][0])))))
    rec_v[pl.ds(0, 16)] = rec0
    pltpu.sync_copy(rec_v, shared.at[pl.ds(t * _STR, 16)])
    plsc.subcore_barrier()

    basef = lax.convert_element_type(t * _PPT, jnp.float32)
    lane0 = io == 0
    big = jnp.float32(3e38)

    def read_winner():
        pltpu.sync_copy(shared, all_v)
        best = all_v[pl.ds(0, 16)]
        bv = best[0]
        for tt in range(1, _NT):
            r = all_v[pl.ds(tt * _STR, 16)]
            v = r[0]
            take = v > bv
            bv = jnp.where(take, v, bv)
            best = jnp.where(take, r, best)
        return best[2], best[3], best[4], best[5]

    def emit(i, bx, by, bz, bw):
        iv = jnp.broadcast_to(i, (16,)).astype(jnp.int32)
        for ref, val in ((o0_v, bx), (o1_v, by), (o2_v, bz), (o3_v, bw)):
            plsc.store_scatter(ref, [iv],
                               jnp.broadcast_to(val, (16,)), mask=lane0)

    def fps_iter(i, carry):
        bx, by, bz, bw = read_winner()
        plsc.subcore_barrier()
        emit(i, bx, by, bz, bw)

        def slice_body(s, c2):
            rm, ri = c2
            off = s * 16
            xs = x_v[pl.ds(off, 16)]
            ys = y_v[pl.ds(off, 16)]
            zs = z_v[pl.ds(off, 16)]
            dx = xs - bx
            dy = ys - by
            dz = zs - bz
            d = (dx * dx + dy * dy) + dz * dz
            dm = jnp.minimum(dm_v[pl.ds(off, 16)], d)
            dm_v[pl.ds(off, 16)] = dm
            pf = basef + lax.convert_element_type(off + io, jnp.float32)
            better = dm > rm
            rm = jnp.where(better, dm, rm)
            ri = jnp.where(better, pf, ri)
            return rm, ri

        rm0 = jnp.full((16,), -inf, jnp.float32)
        ri0 = jnp.full((16,), big, jnp.float32)
        rm, ri = lax.fori_loop(0, _SL, slice_body, (rm0, ri0))

        m = jnp.max(rm)
        selif = jnp.min(jnp.where(rm == m, ri, big))
        p = lax.convert_element_type(selif, jnp.int32) - t * _PPT
        pc = jnp.where((p < 0) | (p >= _PPAD), 0, p)
        pvec = jnp.broadcast_to(pc, (16,))
        wx = plsc.load_gather(x_v, [pvec])
        wy = plsc.load_gather(y_v, [pvec])
        wz = plsc.load_gather(z_v, [pvec])
        ww = plsc.load_gather(w_v, [pvec])
        rec = jnp.where(io == 0, m,
              jnp.where(io == 1, selif,
              jnp.where(io == 2, wx,
              jnp.where(io == 3, wy,
              jnp.where(io == 4, wz, ww)))))
        rec_v[pl.ds(0, 16)] = rec
        pltpu.sync_copy(rec_v, shared.at[pl.ds(t * _STR, 16)])
        plsc.subcore_barrier()
        return carry

    lax.fori_loop(0, _NUM - 1, fps_iter, 0)

    bx, by, bz, bw = read_winner()
    emit(_NUM - 1, bx, by, bz, bw)

    @pl.when(t == 0)
    def _():
        pltpu.sync_copy(o0_v, out_hbm.at[0])
        pltpu.sync_copy(o1_v, out_hbm.at[1])
        pltpu.sync_copy(o2_v, out_hbm.at[2])
        pltpu.sync_copy(o3_v, out_hbm.at[3])


@functools.partial(
    pl.kernel,
    out_type=jax.ShapeDtypeStruct((4, _NUM), jnp.float32),
    mesh=plsc.VectorSubcoreMesh(core_axis_name="c", subcore_axis_name="s",
                                num_cores=1, num_subcores=_NT),
    compiler_params=pltpu.CompilerParams(needs_layout_passes=False),
    scratch_types=[
        pltpu.VMEM((_PPAD,), jnp.float32),
        pltpu.VMEM((_PPAD,), jnp.float32),
        pltpu.VMEM((_PPAD,), jnp.float32),
        pltpu.VMEM((_PPAD,), jnp.float32),
        pltpu.VMEM((_PPAD,), jnp.float32),
        pltpu.VMEM((16,), jnp.float32),
        pltpu.VMEM((_NT * _STR,), jnp.float32),
        pltpu.VMEM((_NUM,), jnp.float32),
        pltpu.VMEM((_NUM,), jnp.float32),
        pltpu.VMEM((_NUM,), jnp.float32),
        pltpu.VMEM((_NUM,), jnp.float32),
        pltpu.VMEM_SHARED((_NT * _STR,), jnp.float32),
    ],
)
def _fps_sc(x_hbm, y_hbm, z_hbm, w_hbm, out_hbm, *scratch):
    _sc_body(x_hbm, y_hbm, z_hbm, w_hbm, out_hbm, *scratch)


def kernel(pcd):
    xyz = pcd[:, :3]
    xyz = xyz - jnp.mean(xyz, axis=0, keepdims=True)
    dis = jnp.linalg.norm(xyz, axis=1)
    max_dis = jnp.max(dis)
    xyz = xyz / max_dis
    pcdn = pcd.at[:, :3].set(xyz)

    chans = pcdn.T.reshape(4, _NT, _PPT)
    chans = jnp.pad(chans, ((0, 0), (0, 0), (0, _PPAD - _PPT)))
    return _fps_sc(chans[0], chans[1], chans[2], chans[3])


# speedup vs baseline: 1.6212x; 1.6212x over previous
import functools

import jax
import jax.numpy as jnp
from jax import lax
from jax.experimental import pallas as pl
from jax.experimental.pallas import tpu as pltpu
from jax.experimental.pallas import tpu_sc as plsc

_N = 20000
_NUM = 2048
_NT = 16
_PPT = 1250
_SL = 79
_PPAD = _SL * 16
_STR = 64


def _sc_body(x_hbm, y_hbm, z_hbm, w_hbm, out_hbm,
             x_v, y_v, z_v, w_v, dm_v, rec_v, all_v,
             o0_v, o1_v, o2_v, o3_v, shared):
    t = lax.axis_index("s")
    inf = jnp.float32(jnp.inf)
    io = lax.iota(jnp.int32, 16)

    pltpu.sync_copy(x_hbm.at[t], x_v)
    pltpu.sync_copy(y_hbm.at[t], y_v)
    pltpu.sync_copy(z_hbm.at[t], z_v)
    pltpu.sync_copy(w_hbm.at[t], w_v)

    def init_dm(s, carry):
        p = s * 16 + io
        dm_v[pl.ds(s * 16, 16)] = jnp.where(p < _PPT, inf, -inf)
        return carry

    lax.fori_loop(0, _SL, init_dm, 0)

    v0 = jnp.where(t == 0, inf, -inf)
    rec0 = jnp.where(io == 0, v0,
           jnp.where(io == 1, jnp.float32(0.0),
           jnp.where(io == 2, x_v[pl.ds(0, 16)][0],
           jnp.where(io == 3, y_v[pl.ds(0, 16)][0],
           jnp.where(io == 4, z_v[pl.ds(0, 16)][0],
                     w_v[pl.ds(0, 16)][0])))))
    rec_v[pl.ds(0, 16)] = rec0
    pltpu.sync_copy(rec_v, shared.at[pl.ds(t * _STR, 16)])
    plsc.subcore_barrier()

    basef = lax.convert_element_type(t * _PPT, jnp.float32)
    lane0 = io == 0
    big = jnp.float32(3e38)

    def read_winner():
        pltpu.sync_copy(shared, all_v)
        vals = plsc.load_gather(all_v, [io * _STR])
        m = jnp.max(vals)
        tid = jnp.min(jnp.where(vals == m, io, _NT))
        r = plsc.load_gather(all_v, [jnp.broadcast_to(tid * _STR, (16,)) + io])
        return r[2], r[3], r[4], r[5]

    def emit(i, bx, by, bz, bw):
        iv = jnp.broadcast_to(i, (16,)).astype(jnp.int32)
        for ref, val in ((o0_v, bx), (o1_v, by), (o2_v, bz), (o3_v, bw)):
            plsc.store_scatter(ref, [iv],
                               jnp.broadcast_to(val, (16,)), mask=lane0)

    def fps_iter(i, carry):
        bx, by, bz, bw = read_winner()
        plsc.subcore_barrier()
        emit(i, bx, by, bz, bw)

        iof = lax.convert_element_type(io, jnp.float32)
        rm = jnp.full((16,), -inf, jnp.float32)
        ri = jnp.full((16,), big, jnp.float32)
        for sl in range(_SL):
            off = sl * 16
            xs = x_v[pl.ds(off, 16)]
            ys = y_v[pl.ds(off, 16)]
            zs = z_v[pl.ds(off, 16)]
            dx = xs - bx
            dy = ys - by
            dz = zs - bz
            d = (dx * dx + dy * dy) + dz * dz
            dm = jnp.minimum(dm_v[pl.ds(off, 16)], d)
            dm_v[pl.ds(off, 16)] = dm
            pf = basef + (iof + float(off))
            better = dm > rm
            rm = jnp.where(better, dm, rm)
            ri = jnp.where(better, pf, ri)

        m = jnp.max(rm)
        selif = jnp.min(jnp.where(rm == m, ri, big))
        p = lax.convert_element_type(selif, jnp.int32) - t * _PPT
        pc = jnp.where((p < 0) | (p >= _PPAD), 0, p)
        pvec = jnp.broadcast_to(pc, (16,))
        wx = plsc.load_gather(x_v, [pvec])
        wy = plsc.load_gather(y_v, [pvec])
        wz = plsc.load_gather(z_v, [pvec])
        ww = plsc.load_gather(w_v, [pvec])
        rec = jnp.where(io == 0, m,
              jnp.where(io == 1, selif,
              jnp.where(io == 2, wx,
              jnp.where(io == 3, wy,
              jnp.where(io == 4, wz, ww)))))
        rec_v[pl.ds(0, 16)] = rec
        pltpu.sync_copy(rec_v, shared.at[pl.ds(t * _STR, 16)])
        plsc.subcore_barrier()
        return carry

    lax.fori_loop(0, _NUM - 1, fps_iter, 0)

    bx, by, bz, bw = read_winner()
    emit(_NUM - 1, bx, by, bz, bw)

    @pl.when(t == 0)
    def _():
        pltpu.sync_copy(o0_v, out_hbm.at[0])
        pltpu.sync_copy(o1_v, out_hbm.at[1])
        pltpu.sync_copy(o2_v, out_hbm.at[2])
        pltpu.sync_copy(o3_v, out_hbm.at[3])


@functools.partial(
    pl.kernel,
    out_type=jax.ShapeDtypeStruct((4, _NUM), jnp.float32),
    mesh=plsc.VectorSubcoreMesh(core_axis_name="c", subcore_axis_name="s",
                                num_cores=1, num_subcores=_NT),
    compiler_params=pltpu.CompilerParams(needs_layout_passes=False),
    scratch_types=[
        pltpu.VMEM((_PPAD,), jnp.float32),
        pltpu.VMEM((_PPAD,), jnp.float32),
        pltpu.VMEM((_PPAD,), jnp.float32),
        pltpu.VMEM((_PPAD,), jnp.float32),
        pltpu.VMEM((_PPAD,), jnp.float32),
        pltpu.VMEM((16,), jnp.float32),
        pltpu.VMEM((_NT * _STR,), jnp.float32),
        pltpu.VMEM((_NUM,), jnp.float32),
        pltpu.VMEM((_NUM,), jnp.float32),
        pltpu.VMEM((_NUM,), jnp.float32),
        pltpu.VMEM((_NUM,), jnp.float32),
        pltpu.VMEM_SHARED((_NT * _STR,), jnp.float32),
    ],
)
def _fps_sc(x_hbm, y_hbm, z_hbm, w_hbm, out_hbm, *scratch):
    _sc_body(x_hbm, y_hbm, z_hbm, w_hbm, out_hbm, *scratch)


def kernel(pcd):
    xyz = pcd[:, :3]
    xyz = xyz - jnp.mean(xyz, axis=0, keepdims=True)
    dis = jnp.linalg.norm(xyz, axis=1)
    max_dis = jnp.max(dis)
    xyz = xyz / max_dis
    pcdn = pcd.at[:, :3].set(xyz)

    chans = pcdn.T.reshape(4, _NT, _PPT)
    chans = jnp.pad(chans, ((0, 0), (0, 0), (0, _PPAD - _PPT)))
    return _fps_sc(chans[0], chans[1], chans[2], chans[3])
